# baseline (speedup 1.0000x reference)
import functools

import jax
import jax.numpy as jnp
from jax.experimental import pallas as pl
from jax.experimental.pallas import tpu as pltpu

FEAT = 128
G = 16
PROP_ITER = 4


def _softsign(v):
    return v / (1.0 + jnp.abs(v))


def _mm(a, b):
    return jax.lax.dot_general(a, b, (((1,), (1,)), ((), ())),
                               preferred_element_type=jnp.float32)


def _pbuild_body(h_ref, w1_ref, b1_ref, w2_ref, b2_ref, w3_ref, b3_ref, p_ref):
    h = h_ref[...]
    g = _softsign(_mm(h, w1_ref[...]) + b1_ref[...])
    g = _softsign(_mm(g, w2_ref[...]) + b2_ref[...])
    g = _mm(g, w3_ref[...]) + b3_ref[...]
    g = jnp.clip(g, -25.0, 25.0)
    eg = jnp.exp(g)
    p_ref[:, :FEAT] = eg * h
    p_ref[:, FEAT:] = eg


def _prop_body(a_ref, p_ref, h_ref, wih_ref, bih_ref, whh_ref, bhh_ref, out_ref):
    a = a_ref[...].astype(jnp.float32)
    m = jax.lax.dot_general(a, p_ref[...], (((1,), (0,)), ((), ())),
                            preferred_element_type=jnp.float32)
    aggr = m[:, :FEAT] / (m[:, FEAT:] + 1e-16)
    h = h_ref[...]
    gi = _mm(aggr, wih_ref[...]) + bih_ref[...]
    gh = _mm(h, whh_ref[...]) + bhh_ref[...]
    r = jax.nn.sigmoid(gi[:, :FEAT] + gh[:, :FEAT])
    z = jax.nn.sigmoid(gi[:, FEAT:2 * FEAT] + gh[:, FEAT:2 * FEAT])
    n = jnp.tanh(gi[:, 2 * FEAT:] + r * gh[:, 2 * FEAT:])
    out_ref[...] = (1.0 - z) * n + z * h


def _final_body(h_ref, x_ref, w1h_ref, w1x_ref, b1_ref, w2_ref, b2_ref,
                wj_ref, bj_ref, oh_ref, ow1_ref, ob1_ref, ow2_ref, ob2_ref,
                ow3_ref, ob3_ref, out_ref, acc_ref):
    i = pl.program_id(0)
    h = h_ref[...]
    x = x_ref[...]
    a = _softsign(_mm(h, w1h_ref[...]) + _mm(x, w1x_ref[...]) + b1_ref[...])
    a = _softsign(_mm(a, w2_ref[...]) + b2_ref[...])
    a = a - jnp.max(a, axis=1, keepdims=True)
    e = jnp.exp(a)
    a = e / jnp.sum(e, axis=1, keepdims=True)
    nj = _softsign(_mm(x, wj_ref[...]) + bj_ref[...])
    prop = a * nj
    contrib = jax.lax.dot_general(oh_ref[...], prop, (((0,), (0,)), ((), ())),
                                  preferred_element_type=jnp.float32)

    @pl.when(i == 0)
    def _():
        acc_ref[...] = contrib

    @pl.when(i > 0)
    def _():
        acc_ref[...] = acc_ref[...] + contrib

    @pl.when(i == pl.num_programs(0) - 1)
    def _():
        o = jax.nn.relu(_mm(acc_ref[...], ow1_ref[...]) + ob1_ref[...])
        o = jax.nn.relu(_mm(o, ow2_ref[...]) + ob2_ref[...])
        out_ref[...] = _mm(o, ow3_ref[...]) + ob3_ref[...]


def kernel(x, edge_index, edge_attr, batch, t, gate_w1, gate_b1, gate_w2,
           gate_b2, gate_w3, gate_b3, gru_wih, gru_whh, gru_bih, gru_bhh,
           atti_w1, atti_b1, atti_w2, atti_b2, attj_w1, attj_b1, out_w1,
           out_b1, out_w2, out_b2, out_w3, out_b3):
    n = x.shape[0]
    br = 200 if n % 200 == 0 else 8
    grid = n // br

    row, col = edge_index[0], edge_index[1]
    mask = (edge_attr <= t[0]).astype(jnp.float32)
    row2 = jnp.concatenate([row, col])
    col2 = jnp.concatenate([col, row])
    val2 = jnp.concatenate([mask, mask])
    adj = jnp.zeros((n, n), jnp.bfloat16).at[col2, row2].max(
        val2.astype(jnp.bfloat16))

    row_spec = pl.BlockSpec((br, FEAT), lambda i: (i, 0))
    full = lambda s: pl.BlockSpec(s, lambda i: tuple(0 for _ in s))

    def w_specs(*shapes):
        return [full(s) for s in shapes]

    b1 = gate_b1.reshape(1, -1)
    b2 = gate_b2.reshape(1, -1)
    w3 = jnp.tile(gate_w3, (FEAT, 1))
    b3 = jnp.broadcast_to(gate_b3.reshape(1, 1), (1, FEAT))
    bih = gru_bih.reshape(1, -1)
    bhh = gru_bhh.reshape(1, -1)

    pbuild = pl.pallas_call(
        _pbuild_body,
        grid=(grid,),
        in_specs=[row_spec] + w_specs(gate_w1.shape, b1.shape, gate_w2.shape,
                                      b2.shape, w3.shape, b3.shape),
        out_specs=pl.BlockSpec((br, 2 * FEAT), lambda i: (i, 0)),
        out_shape=jax.ShapeDtypeStruct((n, 2 * FEAT), jnp.float32),
    )

    prop = pl.pallas_call(
        _prop_body,
        grid=(grid,),
        in_specs=[pl.BlockSpec((br, n), lambda i: (i, 0)),
                  full((n, 2 * FEAT)), row_spec] +
                 w_specs(gru_wih.shape, bih.shape, gru_whh.shape, bhh.shape),
        out_specs=row_spec,
        out_shape=jax.ShapeDtypeStruct((n, FEAT), jnp.float32),
    )

    h = x
    for _ in range(PROP_ITER):
        p = pbuild(h, gate_w1, b1, gate_w2, b2, w3, b3)
        h = prop(adj, p, h, gru_wih, bih, gru_whh, bhh)

    w1h = atti_w1[:, :FEAT]
    w1x = atti_w1[:, FEAT:]
    onehot = (batch[:, None] == jnp.arange(G, dtype=batch.dtype)[None, :]
              ).astype(jnp.float32)
    ab1 = atti_b1.reshape(1, -1)
    ab2 = atti_b2.reshape(1, -1)
    bj = attj_b1.reshape(1, -1)
    ob1 = out_b1.reshape(1, -1)
    ob2 = out_b2.reshape(1, -1)
    ob3 = out_b3.reshape(1, -1)

    final = pl.pallas_call(
        _final_body,
        grid=(grid,),
        in_specs=[row_spec, row_spec] +
                 w_specs(w1h.shape, w1x.shape, ab1.shape, atti_w2.shape,
                         ab2.shape, attj_w1.shape, bj.shape) +
                 [pl.BlockSpec((br, G), lambda i: (i, 0))] +
                 w_specs(out_w1.shape, ob1.shape, out_w2.shape, ob2.shape,
                         out_w3.shape, ob3.shape),
        out_specs=pl.BlockSpec((G, FEAT), lambda i: (0, 0)),
        out_shape=jax.ShapeDtypeStruct((G, FEAT), jnp.float32),
        scratch_shapes=[pltpu.VMEM((G, FEAT), jnp.float32)],
    )
    return final(h, x, w1h, w1x, ab1, atti_w2, ab2, attj_w1, bj, onehot,
                 out_w1, ob1, out_w2, ob2, out_w3, ob3)


# speedup vs baseline: 3.1524x; 3.1524x over previous
import functools

import jax
import jax.numpy as jnp
from jax.experimental import pallas as pl
from jax.experimental.pallas import tpu as pltpu

FEAT = 128
G = 16
PROP_ITER = 4


def _softsign(v):
    return v / (1.0 + jnp.abs(v))


def _mm(a, b):
    return jax.lax.dot_general(a, b, (((1,), (1,)), ((), ())),
                               preferred_element_type=jnp.float32)


def _pbuild_body(h_ref, w1_ref, b1_ref, w2_ref, b2_ref, w3_ref, b3_ref, p_ref):
    h = h_ref[...]
    g = _softsign(_mm(h, w1_ref[...]) + b1_ref[...])
    g = _softsign(_mm(g, w2_ref[...]) + b2_ref[...])
    g = _mm(g, w3_ref[...]) + b3_ref[...]
    g = jnp.clip(g, -25.0, 25.0)
    eg = jnp.exp(g)
    p_ref[:, :FEAT] = eg * h
    p_ref[:, FEAT:] = eg


def _prop_body(a_ref, p_ref, h_ref, wih_ref, bih_ref, whh_ref, bhh_ref, out_ref):
    m = jax.lax.dot_general(a_ref[...], p_ref[...], (((1,), (0,)), ((), ())),
                            preferred_element_type=jnp.float32)
    aggr = m[:, :FEAT] / (m[:, FEAT:] + 1e-16)
    h = h_ref[...]
    gi = _mm(aggr, wih_ref[...]) + bih_ref[...]
    gh = _mm(h, whh_ref[...]) + bhh_ref[...]
    r = jax.nn.sigmoid(gi[:, :FEAT] + gh[:, :FEAT])
    z = jax.nn.sigmoid(gi[:, FEAT:2 * FEAT] + gh[:, FEAT:2 * FEAT])
    n = jnp.tanh(gi[:, 2 * FEAT:] + r * gh[:, 2 * FEAT:])
    out_ref[...] = (1.0 - z) * n + z * h


def _final_body(h_ref, x_ref, w1h_ref, w1x_ref, b1_ref, w2_ref, b2_ref,
                wj_ref, bj_ref, oh_ref, ow1_ref, ob1_ref, ow2_ref, ob2_ref,
                ow3_ref, ob3_ref, out_ref, acc_ref):
    i = pl.program_id(0)
    h = h_ref[...]
    x = x_ref[...]
    a = _softsign(_mm(h, w1h_ref[...]) + _mm(x, w1x_ref[...]) + b1_ref[...])
    a = _softsign(_mm(a, w2_ref[...]) + b2_ref[...])
    a = a - jnp.max(a, axis=1, keepdims=True)
    e = jnp.exp(a)
    a = e / jnp.sum(e, axis=1, keepdims=True)
    nj = _softsign(_mm(x, wj_ref[...]) + bj_ref[...])
    prop = a * nj
    contrib = jax.lax.dot_general(oh_ref[...], prop, (((0,), (0,)), ((), ())),
                                  preferred_element_type=jnp.float32)

    @pl.when(i == 0)
    def _():
        acc_ref[...] = contrib

    @pl.when(i > 0)
    def _():
        acc_ref[...] = acc_ref[...] + contrib

    @pl.when(i == pl.num_programs(0) - 1)
    def _():
        o = jax.nn.relu(_mm(acc_ref[...], ow1_ref[...]) + ob1_ref[...])
        o = jax.nn.relu(_mm(o, ow2_ref[...]) + ob2_ref[...])
        out_ref[...] = _mm(o, ow3_ref[...]) + ob3_ref[...]


def kernel(x, edge_index, edge_attr, batch, t, gate_w1, gate_b1, gate_w2,
           gate_b2, gate_w3, gate_b3, gru_wih, gru_whh, gru_bih, gru_bhh,
           atti_w1, atti_b1, atti_w2, atti_b2, attj_w1, attj_b1, out_w1,
           out_b1, out_w2, out_b2, out_w3, out_b3):
    n = x.shape[0]
    br = 200 if n % 200 == 0 else 8
    grid = n // br

    row, col = edge_index[0], edge_index[1]
    mask = (edge_attr <= t[0]).astype(jnp.float32)
    row2 = jnp.concatenate([row, col])
    col2 = jnp.concatenate([col, row])
    val2 = jnp.concatenate([mask, mask])
    adj = jnp.zeros((n, n), jnp.float32).at[col2, row2].max(val2)

    row_spec = pl.BlockSpec((br, FEAT), lambda i: (i, 0))
    full = lambda s: pl.BlockSpec(s, lambda i: tuple(0 for _ in s))

    def w_specs(*shapes):
        return [full(s) for s in shapes]

    b1 = gate_b1.reshape(1, -1)
    b2 = gate_b2.reshape(1, -1)
    w3 = jnp.tile(gate_w3, (FEAT, 1))
    b3 = jnp.broadcast_to(gate_b3.reshape(1, 1), (1, FEAT))
    bih = gru_bih.reshape(1, -1)
    bhh = gru_bhh.reshape(1, -1)

    pbuild = pl.pallas_call(
        _pbuild_body,
        grid=(grid,),
        in_specs=[row_spec] + w_specs(gate_w1.shape, b1.shape, gate_w2.shape,
                                      b2.shape, w3.shape, b3.shape),
        out_specs=pl.BlockSpec((br, 2 * FEAT), lambda i: (i, 0)),
        out_shape=jax.ShapeDtypeStruct((n, 2 * FEAT), jnp.float32),
    )

    brp = 400 if n % 400 == 0 else br
    row_spec_p = pl.BlockSpec((brp, FEAT), lambda i: (i, 0))
    prop = pl.pallas_call(
        _prop_body,
        grid=(n // brp,),
        in_specs=[pl.BlockSpec((brp, n), lambda i: (i, 0)),
                  full((n, 2 * FEAT)), row_spec_p] +
                 w_specs(gru_wih.shape, bih.shape, gru_whh.shape, bhh.shape),
        out_specs=row_spec_p,
        out_shape=jax.ShapeDtypeStruct((n, FEAT), jnp.float32),
    )

    h = x
    for _ in range(PROP_ITER):
        p = pbuild(h, gate_w1, b1, gate_w2, b2, w3, b3)
        h = prop(adj, p, h, gru_wih, bih, gru_whh, bhh)

    w1h = atti_w1[:, :FEAT]
    w1x = atti_w1[:, FEAT:]
    onehot = (batch[:, None] == jnp.arange(G, dtype=batch.dtype)[None, :]
              ).astype(jnp.float32)
    ab1 = atti_b1.reshape(1, -1)
    ab2 = atti_b2.reshape(1, -1)
    bj = attj_b1.reshape(1, -1)
    ob1 = out_b1.reshape(1, -1)
    ob2 = out_b2.reshape(1, -1)
    ob3 = out_b3.reshape(1, -1)

    final = pl.pallas_call(
        _final_body,
        grid=(grid,),
        in_specs=[row_spec, row_spec] +
                 w_specs(w1h.shape, w1x.shape, ab1.shape, atti_w2.shape,
                         ab2.shape, attj_w1.shape, bj.shape) +
                 [pl.BlockSpec((br, G), lambda i: (i, 0))] +
                 w_specs(out_w1.shape, ob1.shape, out_w2.shape, ob2.shape,
                         out_w3.shape, ob3.shape),
        out_specs=pl.BlockSpec((G, FEAT), lambda i: (0, 0)),
        out_shape=jax.ShapeDtypeStruct((G, FEAT), jnp.float32),
        scratch_shapes=[pltpu.VMEM((G, FEAT), jnp.float32)],
    )
    return final(h, x, w1h, w1x, ab1, atti_w2, ab2, attj_w1, bj, onehot,
                 out_w1, ob1, out_w2, ob2, out_w3, ob3)
